# baseline (speedup 1.0000x reference)
import functools

import jax
import jax.numpy as jnp
from jax import lax
from jax.experimental import pallas as pl
from jax.experimental.pallas import tpu as pltpu
from jax.experimental.pallas import tpu_sc as plsc

B, S, D = 8, 512, 768
NC, NS, L = 2, 16, 16
NW = NC * NS
PW = 128
CPP = PW // L
DSC = 256
DTC = D - DSC
PPB = DSC // PW
NPAN = B * PPB
SH = S // 2
UNROLL = 8
TCB = 128


def _exp_poly(x):
    e = 1.0 / 5040.0 + x * (1.0 / 40320.0)
    e = 1.0 / 720.0 + x * e
    e = 1.0 / 120.0 + x * e
    e = 1.0 / 24.0 + x * e
    e = 1.0 / 6.0 + x * e
    e = 0.5 + x * e
    e = 1.0 + x * e
    return 1.0 + x * e


def _sc_body(h_hbm, t_hbm, p_hbm, out_hbm,
             p_v, t_v, r_v, hb_v, sin0, sin1, sout0):
    wid = lax.axis_index("s") * NC + lax.axis_index("c")
    pan = wid % NPAN
    upper = wid >= NPAN
    b = pan // PPB
    d0 = (pan % PPB) * PW

    pltpu.async_copy(h_hbm.at[b, pl.ds(0, SH), pl.ds(d0, PW)],
                     hb_v.at[0], sin0)

    @pl.when(upper)
    def _():
        pltpu.async_copy(h_hbm.at[b, pl.ds(SH, SH), pl.ds(d0, PW)],
                         hb_v.at[1], sin1)

    pltpu.sync_copy(p_hbm, p_v)
    pltpu.sync_copy(t_hbm.at[b], t_v)

    zeros_i = jnp.zeros((L,), jnp.int32)
    iota = lax.iota(jnp.int32, L)
    eps_v = plsc.load_gather(p_v, [zeros_i + 1])
    beta_v = plsc.load_gather(p_v, [zeros_i + 2])

    def build(c, _):
        idx = c * L + iota
        prev = jnp.where(idx > 0, idx - 1, 0)
        dt = t_v[pl.ds(c * L, L)] - plsc.load_gather(t_v, [prev])
        r_v[pl.ds(c * L, L)] = _exp_poly(beta_v * dt)
        return 0
    lax.fori_loop(0, S // L, build, 0)

    zf = jnp.zeros((L,), jnp.float32)
    zero_carry = (zf,) * (2 * CPP)

    def reduce_half(carry):
        hb = hb_v.at[0]

        def step(it, c):
            for u in range(UNROLL):
                s = it * UNROLL + u
                rv = plsc.load_gather(r_v, [jnp.full((L,), s, jnp.int32)])
                hvs = [hb[s, pl.ds(ci * L, L)] for ci in range(CPP)]
                a1s = [c[2 * ci] + hvs[ci] for ci in range(CPP)]
                rps = [jnp.maximum(hvs[ci], 0.0) for ci in range(CPP)]
                eps_rps = [eps_v * rps[ci] for ci in range(CPP)]
                ra2s = [rv * c[2 * ci + 1] for ci in range(CPP)]
                a2s = [ra2s[ci] + eps_rps[ci] for ci in range(CPP)]
                c = tuple(x for ci in range(CPP) for x in (a1s[ci], a2s[ci]))
            return c
        return lax.fori_loop(0, SH // UNROLL, step, carry)

    def scan_half(k, carry):
        hb = hb_v.at[k]

        def step(it, c):
            for u in range(UNROLL):
                s = it * UNROLL + u
                rv = plsc.load_gather(
                    r_v, [jnp.full((L,), k * SH + s, jnp.int32)])
                hvs = [hb[s, pl.ds(ci * L, L)] for ci in range(CPP)]
                a1s = [c[2 * ci] + hvs[ci] for ci in range(CPP)]
                rps = [jnp.maximum(hvs[ci], 0.0) for ci in range(CPP)]
                eps_rps = [eps_v * rps[ci] for ci in range(CPP)]
                ra2s = [rv * c[2 * ci + 1] for ci in range(CPP)]
                a2s = [ra2s[ci] + eps_rps[ci] for ci in range(CPP)]
                for ci in range(CPP):
                    hb[s, pl.ds(ci * L, L)] = a1s[ci] + a2s[ci]
                c = tuple(x for ci in range(CPP) for x in (a1s[ci], a2s[ci]))
            return c
        return lax.fori_loop(0, SH // UNROLL, step, carry)

    @pl.when(jnp.logical_not(upper))
    def _():
        pltpu.make_async_copy(h_hbm.at[b, pl.ds(0, SH), pl.ds(d0, PW)],
                              hb_v.at[0], sin0).wait()
        scan_half(0, zero_carry)
        pltpu.async_copy(hb_v.at[0],
                         out_hbm.at[b, pl.ds(0, SH), pl.ds(d0, PW)], sout0)
        pltpu.make_async_copy(hb_v.at[0],
                              out_hbm.at[b, pl.ds(0, SH), pl.ds(d0, PW)],
                              sout0).wait()

    @pl.when(upper)
    def _():
        pltpu.make_async_copy(h_hbm.at[b, pl.ds(0, SH), pl.ds(d0, PW)],
                              hb_v.at[0], sin0).wait()
        carry = reduce_half(zero_carry)
        pltpu.make_async_copy(h_hbm.at[b, pl.ds(SH, SH), pl.ds(d0, PW)],
                              hb_v.at[1], sin1).wait()
        scan_half(1, carry)
        pltpu.async_copy(hb_v.at[1],
                         out_hbm.at[b, pl.ds(SH, SH), pl.ds(d0, PW)], sout0)
        pltpu.make_async_copy(hb_v.at[1],
                              out_hbm.at[b, pl.ds(SH, SH), pl.ds(d0, PW)],
                              sout0).wait()


def _tc_body(tcol_ref, p_ref, h_ref, out_ref):
    eps = p_ref[0, 0]
    beta = p_ref[0, 1]
    tcol = tcol_ref[0]
    dt = tcol - tcol[0:1, :]
    wd = jnp.exp(-beta * dt)
    wg = eps * jnp.exp(beta * dt)
    hb = h_ref[0]
    z = jnp.concatenate([hb, wd * jnp.maximum(hb, 0.0)], axis=1)

    ri = lax.broadcasted_iota(jnp.int32, (TCB, TCB), 0)
    ci = lax.broadcasted_iota(jnp.int32, (TCB, TCB), 1)
    tri = (ci <= ri).astype(jnp.float32)

    run = jnp.zeros((1, z.shape[1]), jnp.float32)
    chunks = []
    for c in range(S // TCB):
        blk = z[c * TCB:(c + 1) * TCB, :]
        cs = jnp.dot(tri, blk, preferred_element_type=jnp.float32) + run
        run = cs[TCB - 1:TCB, :]
        chunks.append(cs)
    zc = jnp.concatenate(chunks, axis=0)
    w = zc.shape[1] // 2
    out_ref[0] = zc[:, :w] + wg * zc[:, w:]


@jax.jit
def _heat(h, t, params):
    mesh = plsc.VectorSubcoreMesh(core_axis_name="c", subcore_axis_name="s")
    sc = functools.partial(
        pl.kernel,
        out_type=jax.ShapeDtypeStruct((B, S, DSC), jnp.float32),
        mesh=mesh,
        scratch_types=[
            pltpu.VMEM((L,), jnp.float32),
            pltpu.VMEM((S,), jnp.float32),
            pltpu.VMEM((S,), jnp.float32),
            pltpu.VMEM((2, SH, PW), jnp.float32),
            pltpu.SemaphoreType.DMA,
            pltpu.SemaphoreType.DMA,
            pltpu.SemaphoreType.DMA,
        ],
        compiler_params=pltpu.CompilerParams(
            use_tc_tiling_on_sc=True, needs_layout_passes=False,
            skip_device_barrier=True),
    )(_sc_body)
    sc_out = sc(h, t, params)

    tc_full = pl.pallas_call(
        _tc_body,
        grid=(B, 2),
        in_specs=[
            pl.BlockSpec((1, S, 1), lambda b, j: (b, 0, 0)),
            pl.BlockSpec(memory_space=pltpu.SMEM),
            pl.BlockSpec((1, S, DTC // 2), lambda b, j: (b, 0, 1 + j)),
        ],
        out_specs=pl.BlockSpec((1, S, DTC // 2), lambda b, j: (b, 0, 1 + j)),
        out_shape=jax.ShapeDtypeStruct((B, S, D), jnp.float32),
    )(t[:, :, None], params[None, 1:3], h)

    return lax.dynamic_update_slice(tc_full, sc_out, (0, 0, 0))


def kernel(h, t, epsilon, beta):
    params = jnp.zeros((L,), jnp.float32)
    params = params.at[1].set(epsilon).at[2].set(beta)
    return _heat(h.astype(jnp.float32), t.astype(jnp.float32), params)


# speedup vs baseline: 1.1442x; 1.1442x over previous
import functools

import jax
import jax.numpy as jnp
from jax import lax
from jax.experimental import pallas as pl
from jax.experimental.pallas import tpu as pltpu
from jax.experimental.pallas import tpu_sc as plsc

B, S, D = 8, 512, 768
NC, NS, L = 2, 16, 16
NW = NC * NS
PW = 128
CPP = PW // L
DSC = 512
DTC = D - DSC
PPB = DSC // PW
SH = S // 2
UNROLL = 8


def _exp_poly(x):
    e = 1.0 / 5040.0 + x * (1.0 / 40320.0)
    e = 1.0 / 720.0 + x * e
    e = 1.0 / 120.0 + x * e
    e = 1.0 / 24.0 + x * e
    e = 1.0 / 6.0 + x * e
    e = 0.5 + x * e
    e = 1.0 + x * e
    return 1.0 + x * e


def _sc_body(h_hbm, t_hbm, p_hbm, out_hbm,
             p_v, t_v, r_v, hb_v, sin0, sin1, sout0, sout1):
    wid = lax.axis_index("s") * NC + lax.axis_index("c")
    b = wid // PPB
    d0 = (wid % PPB) * PW

    srcs = [h_hbm.at[b, pl.ds(k * SH, SH), pl.ds(d0, PW)] for k in (0, 1)]
    dsts = [out_hbm.at[b, pl.ds(k * SH, SH), pl.ds(d0, PW)] for k in (0, 1)]
    sins = [sin0, sin1]
    souts = [sout0, sout1]

    for k in (0, 1):
        pltpu.async_copy(srcs[k], hb_v.at[k], sins[k])

    pltpu.sync_copy(p_hbm, p_v)
    pltpu.sync_copy(t_hbm.at[b], t_v)

    zeros_i = jnp.zeros((L,), jnp.int32)
    iota = lax.iota(jnp.int32, L)
    eps_v = plsc.load_gather(p_v, [zeros_i + 1])
    beta_v = plsc.load_gather(p_v, [zeros_i + 2])

    def build(c, _):
        idx = c * L + iota
        prev = jnp.where(idx > 0, idx - 1, 0)
        dt = t_v[pl.ds(c * L, L)] - plsc.load_gather(t_v, [prev])
        r_v[pl.ds(c * L, L)] = _exp_poly(beta_v * dt)
        return 0
    lax.fori_loop(0, S // L, build, 0)

    zf = jnp.zeros((L,), jnp.float32)

    def half(k, carry):
        hb = hb_v.at[k]

        @pl.when(k == 0)
        def _():
            pltpu.make_async_copy(srcs[0], hb_v.at[0], sins[0]).wait()

        @pl.when(k == 1)
        def _():
            pltpu.make_async_copy(srcs[1], hb_v.at[1], sins[1]).wait()

        def step(it, c):
            for u in range(UNROLL):
                s = it * UNROLL + u
                rv = plsc.load_gather(
                    r_v, [jnp.full((L,), k * SH + s, jnp.int32)])
                hvs = [hb[s, pl.ds(ci * L, L)] for ci in range(CPP)]
                a1s = [c[2 * ci] + hvs[ci] for ci in range(CPP)]
                rps = [jnp.maximum(hvs[ci], 0.0) for ci in range(CPP)]
                eps_rps = [eps_v * rps[ci] for ci in range(CPP)]
                ra2s = [rv * c[2 * ci + 1] for ci in range(CPP)]
                a2s = [ra2s[ci] + eps_rps[ci] for ci in range(CPP)]
                for ci in range(CPP):
                    hb[s, pl.ds(ci * L, L)] = a1s[ci] + a2s[ci]
                c = tuple(x for ci in range(CPP) for x in (a1s[ci], a2s[ci]))
            return c
        carry = lax.fori_loop(0, SH // UNROLL, step, carry)

        @pl.when(k == 0)
        def _():
            pltpu.async_copy(hb_v.at[0], dsts[0], souts[0])

        @pl.when(k == 1)
        def _():
            pltpu.async_copy(hb_v.at[1], dsts[1], souts[1])
        return carry

    lax.fori_loop(0, 2, half, (zf,) * (2 * CPP))

    pltpu.make_async_copy(hb_v.at[0], dsts[0], souts[0]).wait()
    pltpu.make_async_copy(hb_v.at[1], dsts[1], souts[1]).wait()


def _tc_body(tcol_ref, p_ref, h_ref, out_ref):
    eps = p_ref[0, 0]
    beta = p_ref[0, 1]
    tcol = tcol_ref[0]
    dt = tcol - tcol[0:1, :]
    wd = jnp.exp(-beta * dt)
    wg = eps * jnp.exp(beta * dt)
    hb = h_ref[0]
    z = jnp.concatenate([hb, wd * jnp.maximum(hb, 0.0)], axis=1)
    k = 1
    while k < S:
        z = z + jnp.concatenate(
            [jnp.zeros((k, 2 * DTC), z.dtype), z[:-k]], axis=0)
        k *= 2
    out_ref[0] = z[:, :DTC] + wg * z[:, DTC:]


@jax.jit
def _heat(h, t, params):
    mesh = plsc.VectorSubcoreMesh(core_axis_name="c", subcore_axis_name="s")
    sc = functools.partial(
        pl.kernel,
        out_type=jax.ShapeDtypeStruct((B, S, D), jnp.float32),
        mesh=mesh,
        scratch_types=[
            pltpu.VMEM((L,), jnp.float32),
            pltpu.VMEM((S,), jnp.float32),
            pltpu.VMEM((S,), jnp.float32),
            pltpu.VMEM((2, SH, PW), jnp.float32),
            pltpu.SemaphoreType.DMA,
            pltpu.SemaphoreType.DMA,
            pltpu.SemaphoreType.DMA,
            pltpu.SemaphoreType.DMA,
        ],
        compiler_params=pltpu.CompilerParams(
            use_tc_tiling_on_sc=True, needs_layout_passes=False,
            skip_device_barrier=True),
    )(_sc_body)
    sc_full = sc(h, t, params)

    tc_out = pl.pallas_call(
        _tc_body,
        grid=(B,),
        in_specs=[
            pl.BlockSpec((1, S, 1), lambda b: (b, 0, 0)),
            pl.BlockSpec(memory_space=pltpu.SMEM),
            pl.BlockSpec((1, S, DTC), lambda b: (b, 0, DSC // DTC)),
        ],
        out_specs=pl.BlockSpec((1, S, DTC), lambda b: (b, 0, 0)),
        out_shape=jax.ShapeDtypeStruct((B, S, DTC), jnp.float32),
    )(t[:, :, None], params[None, 1:3], h)

    return lax.dynamic_update_slice(sc_full, tc_out, (0, 0, DSC))


def kernel(h, t, epsilon, beta):
    params = jnp.zeros((L,), jnp.float32)
    params = params.at[1].set(epsilon).at[2].set(beta)
    return _heat(h.astype(jnp.float32), t.astype(jnp.float32), params)
